# baseline (speedup 1.0000x reference)
import functools
import jax
import jax.numpy as jnp
from jax.experimental import pallas as pl
from jax.experimental.pallas import tpu as pltpu

FEAT = 256
KWAY = 5
NSHOT = 5
K1 = 13
K2 = 2
MARGIN = 0.2
BIN = 8
NTOT = 100
NS = KWAY * NSHOT
NQ = NTOT - NS
L = KWAY + NQ
NC = 20
XT = 128


def _matmul(a, b):
    return jax.lax.dot_general(a, b, (((1,), (1,)), ((), ())),
                               preferred_element_type=jnp.float32)


def _safe_sqrt_d2(d2):
    d2 = jnp.maximum(d2, 0.0)
    safe = jnp.where(d2 < 1e-12, 1.0, d2)
    return jnp.where(d2 < 1e-12, 0.0, jnp.sqrt(safe))


def _topk_onehot_gather(dist, table, k):
    nrow, ncol = dist.shape
    col = jax.lax.broadcasted_iota(jnp.int32, (nrow, ncol), 1)
    out = []
    d = dist
    for _ in range(k):
        m = jnp.min(d, axis=1, keepdims=True)
        eq = d == m
        idx = jnp.min(jnp.where(eq, col, ncol), axis=1, keepdims=True)
        oh = col == idx
        out.append(jax.lax.dot_general(
            oh.astype(jnp.float32), table, (((1,), (0,)), ((), ())),
            preferred_element_type=jnp.float32))
        d = jnp.where(oh, 1e30, d)
    return out


def _mlp_combine(planes, w1_ref, b1_ref, w2_ref, b2_ref, nin, nhid):
    h = []
    for o in range(nhid):
        acc = b1_ref[0, o]
        for c in range(nin):
            acc = acc + w1_ref[o, c] * planes[c]
        h.append(jnp.maximum(acc, 0.0))
    z = []
    for c in range(nin):
        acc = b2_ref[0, c]
        for o in range(nhid):
            acc = acc + w2_ref[c, o] * h[o]
        z.append(acc)
    zmax = z[0]
    for c in range(1, nin):
        zmax = jnp.maximum(zmax, z[c])
    e = [jnp.exp(zc - zmax) for zc in z]
    s = e[0]
    for c in range(1, nin):
        s = s + e[c]
    out = planes[0] * (e[0] / s)
    for c in range(1, nin):
        out = out + planes[c] * (e[c] / s)
    return out


def _episode_kernel(inpt_ref, wqk_ref, bqk_ref,
                    sw1_ref, sb1_ref, sw2_ref, sb2_ref,
                    qw1_ref, qb1_ref, qw2_ref, qb2_ref,
                    ypred_ref, loss_ref,
                    q_s, k_s, v_s, s_s, e_s):
    b = pl.program_id(0)
    x = inpt_ref[0]

    feat = _matmul(x, wqk_ref[...]) + bqk_ref[...]
    q_s[...] = feat[:, :FEAT]
    k_s[...] = feat[:, FEAT:]

    for xt in range(FEAT // XT):
        for c in range(NTOT // NC):
            qc = q_s[c * NC:(c + 1) * NC, :]
            kc = k_s[c * NC:(c + 1) * NC, xt * XT:(xt + 1) * XT]
            e = jnp.exp(qc[:, :, None] * kc[:, None, :])
            e_s[c * NC:(c + 1) * NC, :, :] = e
            if c == 0:
                s_s[...] = jnp.sum(e, axis=0)
            else:
                s_s[...] = s_s[...] + jnp.sum(e, axis=0)
        s_s[...] = 1.0 / s_s[...]
        for c in range(NTOT // NC):
            xc = inpt_ref[0, c * NC:(c + 1) * NC, :]
            e = e_s[c * NC:(c + 1) * NC, :, :] * s_s[...][None]
            vc = jnp.sum(e * xc[:, :, None], axis=1)
            v_s[c * NC:(c + 1) * NC, xt * XT:(xt + 1) * XT] = \
                vc + xc[:, xt * XT:(xt + 1) * XT]

    sup = v_s[0:NS, :]
    que = v_s[pl.ds(NS, NQ), :]

    ns2 = jnp.sum(sup * sup, axis=1, keepdims=True)
    nq2 = jnp.sum(que * que, axis=1, keepdims=True)
    pn = jnp.sqrt(ns2) * jnp.transpose(jnp.sqrt(nq2))
    pn = jnp.maximum(pn, 1e-6)
    dist = -(_matmul(sup, que) / pn)

    nb = _topk_onehot_gather(dist, que, K1)
    sfeat = _mlp_combine([sup] + nb, sw1_ref, sb1_ref, sw2_ref, sb2_ref,
                         K1 + 1, 16)

    nb2 = _topk_onehot_gather(jnp.transpose(dist), sup, K2)
    qfeat = _mlp_combine([que] + nb2, qw1_ref, qb1_ref, qw2_ref, qb2_ref,
                         K2 + 1, 16)

    protos = jnp.concatenate(
        [jnp.mean(sfeat[5 * c:5 * c + 5, :], axis=0, keepdims=True)
         for c in range(KWAY)], axis=0)

    f = jnp.concatenate([protos, qfeat], axis=0)
    g = _matmul(f, f)
    na2 = jnp.sum(f * f, axis=1, keepdims=True)
    d2 = na2 + jnp.transpose(na2) - 2.0 * g
    dmat = _safe_sqrt_d2(d2)
    r = jax.lax.broadcasted_iota(jnp.int32, (L, L), 0)
    c = jax.lax.broadcasted_iota(jnp.int32, (L, L), 1)
    eye = r == c
    dii = jnp.sum(jnp.where(eye, dmat, 0.0), axis=1, keepdims=True)
    flm = jnp.where(eye, 0.0, jnp.maximum(MARGIN + dii - dmat, 0.0))
    tot = jnp.sum(flm)
    num = jnp.sum(jnp.where(flm != 0.0, 1.0, 0.0))
    mean_b = jnp.where(num == 0.0, 0.0, tot / jnp.where(num == 0.0, 1.0, num))

    np2 = jnp.sum(protos * protos, axis=1, keepdims=True)
    qf2 = jnp.sum(qfeat * qfeat, axis=1, keepdims=True)
    d2q = qf2 + jnp.transpose(np2) - 2.0 * _matmul(qfeat, protos)
    dq = _safe_sqrt_d2(d2q)

    @pl.when(b == 0)
    def _():
        ypred_ref[...] = dq
        loss_ref[...] = jnp.broadcast_to(mean_b / BIN, (1, 1))

    @pl.when(b > 0)
    def _():
        ypred_ref[...] = ypred_ref[...] + dq
        loss_ref[...] = loss_ref[...] + mean_b / BIN

    @pl.when(b == BIN - 1)
    def _():
        acc = -ypred_ref[...] / BIN
        m = jnp.max(acc, axis=1, keepdims=True)
        e = jnp.exp(acc - m)
        ypred_ref[...] = e / jnp.sum(e, axis=1, keepdims=True)


@jax.jit
def kernel(inpt, label, W_qk, b_qk, sw1, sb1, sw2, sb2, qw1, qb1, qw2, qb2):
    del label
    smem = functools.partial(pl.BlockSpec, memory_space=pltpu.SMEM)
    ypred, loss = pl.pallas_call(
        _episode_kernel,
        grid=(BIN,),
        in_specs=[
            pl.BlockSpec((1, NTOT, FEAT), lambda b: (b, 0, 0)),
            pl.BlockSpec((2 * FEAT, FEAT), lambda b: (0, 0)),
            pl.BlockSpec((1, 2 * FEAT), lambda b: (0, 0)),
            smem((16, K1 + 1), lambda b: (0, 0)),
            smem((1, 16), lambda b: (0, 0)),
            smem((K1 + 1, 16), lambda b: (0, 0)),
            smem((1, K1 + 1), lambda b: (0, 0)),
            smem((16, K2 + 1), lambda b: (0, 0)),
            smem((1, 16), lambda b: (0, 0)),
            smem((K2 + 1, 16), lambda b: (0, 0)),
            smem((1, K2 + 1), lambda b: (0, 0)),
        ],
        out_specs=[
            pl.BlockSpec((NQ, KWAY), lambda b: (0, 0)),
            pl.BlockSpec((1, 1), lambda b: (0, 0)),
        ],
        out_shape=[
            jax.ShapeDtypeStruct((NQ, KWAY), jnp.float32),
            jax.ShapeDtypeStruct((1, 1), jnp.float32),
        ],
        scratch_shapes=[
            pltpu.VMEM((NTOT, FEAT), jnp.float32),
            pltpu.VMEM((NTOT, FEAT), jnp.float32),
            pltpu.VMEM((NTOT, FEAT), jnp.float32),
            pltpu.VMEM((FEAT, XT), jnp.float32),
            pltpu.VMEM((NTOT, FEAT, XT), jnp.float32),
        ],
    )(inpt, W_qk, b_qk.reshape(1, -1),
      sw1, sb1.reshape(1, -1), sw2, sb2.reshape(1, -1),
      qw1, qb1.reshape(1, -1), qw2, qb2.reshape(1, -1))
    return ypred, loss.reshape(())


# speedup vs baseline: 1.2196x; 1.2196x over previous
import functools
import jax
import jax.numpy as jnp
from jax.experimental import pallas as pl
from jax.experimental.pallas import tpu as pltpu

FEAT = 256
KWAY = 5
NSHOT = 5
K1 = 13
K2 = 2
MARGIN = 0.2
BIN = 8
NTOT = 100
NS = KWAY * NSHOT
NQ = NTOT - NS
L = KWAY + NQ
NC = 20
XT = 128


def _matmul(a, b):
    return jax.lax.dot_general(a, b, (((1,), (1,)), ((), ())),
                               preferred_element_type=jnp.float32)


def _safe_sqrt_d2(d2):
    d2 = jnp.maximum(d2, 0.0)
    safe = jnp.where(d2 < 1e-12, 1.0, d2)
    return jnp.where(d2 < 1e-12, 0.0, jnp.sqrt(safe))


def _topk_onehot_gather(dist, table, k):
    nrow, ncol = dist.shape
    col = jax.lax.broadcasted_iota(jnp.int32, (nrow, ncol), 1)
    out = []
    d = dist
    for _ in range(k):
        m = jnp.min(d, axis=1, keepdims=True)
        eq = d == m
        idx = jnp.min(jnp.where(eq, col, ncol), axis=1, keepdims=True)
        oh = col == idx
        out.append(jax.lax.dot_general(
            oh.astype(jnp.float32), table, (((1,), (0,)), ((), ())),
            preferred_element_type=jnp.float32))
        d = jnp.where(oh, 1e30, d)
    return out


def _mlp_combine(planes, w1_ref, b1_ref, w2_ref, b2_ref, nin, nhid):
    h = []
    for o in range(nhid):
        acc = b1_ref[0, o]
        for c in range(nin):
            acc = acc + w1_ref[o, c] * planes[c]
        h.append(jnp.maximum(acc, 0.0))
    z = []
    for c in range(nin):
        acc = b2_ref[0, c]
        for o in range(nhid):
            acc = acc + w2_ref[c, o] * h[o]
        z.append(acc)
    zmax = z[0]
    for c in range(1, nin):
        zmax = jnp.maximum(zmax, z[c])
    e = [jnp.exp(zc - zmax) for zc in z]
    s = e[0]
    for c in range(1, nin):
        s = s + e[c]
    out = planes[0] * (e[0] / s)
    for c in range(1, nin):
        out = out + planes[c] * (e[c] / s)
    return out


def _episode_kernel(inpt_ref, wqk_ref, bqk_ref,
                    sw1_ref, sb1_ref, sw2_ref, sb2_ref,
                    qw1_ref, qb1_ref, qw2_ref, qb2_ref,
                    ypred_ref, loss_ref,
                    q_s, k_s, v_s, s_s):
    b = pl.program_id(0)
    x = inpt_ref[0]

    feat = _matmul(x, wqk_ref[...]) + bqk_ref[...]
    q_s[...] = feat[:, :FEAT]
    k_s[...] = feat[:, FEAT:]

    for c in range(NTOT // NC):
        qc = q_s[c * NC:(c + 1) * NC, :]
        kc = k_s[c * NC:(c + 1) * NC, :]
        p = qc[:, :, None] * kc[:, None, :]
        if c == 0:
            s_s[...] = jnp.sum(jnp.exp(p), axis=0)
        else:
            s_s[...] = s_s[...] + jnp.sum(jnp.exp(p), axis=0)

    s_s[...] = jnp.log(s_s[...])

    for c in range(NTOT // NC):
        qc = q_s[c * NC:(c + 1) * NC, :]
        kc = k_s[c * NC:(c + 1) * NC, :]
        xc = inpt_ref[0, c * NC:(c + 1) * NC, :]
        e = jnp.exp(qc[:, :, None] * kc[:, None, :] - s_s[...][None])
        vc = jnp.sum(e * xc[:, :, None], axis=1)
        v_s[c * NC:(c + 1) * NC, :] = vc + xc

    sup = v_s[0:NS, :]
    que = v_s[pl.ds(NS, NQ), :]

    ns2 = jnp.sum(sup * sup, axis=1, keepdims=True)
    nq2 = jnp.sum(que * que, axis=1, keepdims=True)
    pn = jnp.sqrt(ns2) * jnp.transpose(jnp.sqrt(nq2))
    pn = jnp.maximum(pn, 1e-6)
    dist = -(_matmul(sup, que) / pn)

    nb = _topk_onehot_gather(dist, que, K1)
    sfeat = _mlp_combine([sup] + nb, sw1_ref, sb1_ref, sw2_ref, sb2_ref,
                         K1 + 1, 16)

    nb2 = _topk_onehot_gather(jnp.transpose(dist), sup, K2)
    qfeat = _mlp_combine([que] + nb2, qw1_ref, qb1_ref, qw2_ref, qb2_ref,
                         K2 + 1, 16)

    protos = jnp.concatenate(
        [jnp.mean(sfeat[5 * c:5 * c + 5, :], axis=0, keepdims=True)
         for c in range(KWAY)], axis=0)

    f = jnp.concatenate([protos, qfeat], axis=0)
    g = _matmul(f, f)
    na2 = jnp.sum(f * f, axis=1, keepdims=True)
    d2 = na2 + jnp.transpose(na2) - 2.0 * g
    dmat = _safe_sqrt_d2(d2)
    r = jax.lax.broadcasted_iota(jnp.int32, (L, L), 0)
    c = jax.lax.broadcasted_iota(jnp.int32, (L, L), 1)
    eye = r == c
    dii = jnp.sum(jnp.where(eye, dmat, 0.0), axis=1, keepdims=True)
    flm = jnp.where(eye, 0.0, jnp.maximum(MARGIN + dii - dmat, 0.0))
    tot = jnp.sum(flm)
    num = jnp.sum(jnp.where(flm != 0.0, 1.0, 0.0))
    mean_b = jnp.where(num == 0.0, 0.0, tot / jnp.where(num == 0.0, 1.0, num))

    np2 = jnp.sum(protos * protos, axis=1, keepdims=True)
    qf2 = jnp.sum(qfeat * qfeat, axis=1, keepdims=True)
    d2q = qf2 + jnp.transpose(np2) - 2.0 * _matmul(qfeat, protos)
    dq = _safe_sqrt_d2(d2q)

    @pl.when(b == 0)
    def _():
        ypred_ref[...] = dq
        loss_ref[...] = jnp.broadcast_to(mean_b / BIN, (1, 1))

    @pl.when(b > 0)
    def _():
        ypred_ref[...] = ypred_ref[...] + dq
        loss_ref[...] = loss_ref[...] + mean_b / BIN

    @pl.when(b == BIN - 1)
    def _():
        acc = -ypred_ref[...] / BIN
        m = jnp.max(acc, axis=1, keepdims=True)
        e = jnp.exp(acc - m)
        ypred_ref[...] = e / jnp.sum(e, axis=1, keepdims=True)


@jax.jit
def kernel(inpt, label, W_qk, b_qk, sw1, sb1, sw2, sb2, qw1, qb1, qw2, qb2):
    del label
    smem = functools.partial(pl.BlockSpec, memory_space=pltpu.SMEM)
    ypred, loss = pl.pallas_call(
        _episode_kernel,
        grid=(BIN,),
        in_specs=[
            pl.BlockSpec((1, NTOT, FEAT), lambda b: (b, 0, 0)),
            pl.BlockSpec((2 * FEAT, FEAT), lambda b: (0, 0)),
            pl.BlockSpec((1, 2 * FEAT), lambda b: (0, 0)),
            smem((16, K1 + 1), lambda b: (0, 0)),
            smem((1, 16), lambda b: (0, 0)),
            smem((K1 + 1, 16), lambda b: (0, 0)),
            smem((1, K1 + 1), lambda b: (0, 0)),
            smem((16, K2 + 1), lambda b: (0, 0)),
            smem((1, 16), lambda b: (0, 0)),
            smem((K2 + 1, 16), lambda b: (0, 0)),
            smem((1, K2 + 1), lambda b: (0, 0)),
        ],
        out_specs=[
            pl.BlockSpec((NQ, KWAY), lambda b: (0, 0)),
            pl.BlockSpec((1, 1), lambda b: (0, 0)),
        ],
        out_shape=[
            jax.ShapeDtypeStruct((NQ, KWAY), jnp.float32),
            jax.ShapeDtypeStruct((1, 1), jnp.float32),
        ],
        scratch_shapes=[
            pltpu.VMEM((NTOT, FEAT), jnp.float32),
            pltpu.VMEM((NTOT, FEAT), jnp.float32),
            pltpu.VMEM((NTOT, FEAT), jnp.float32),
            pltpu.VMEM((FEAT, FEAT), jnp.float32),
        ],
    )(inpt, W_qk, b_qk.reshape(1, -1),
      sw1, sb1.reshape(1, -1), sw2, sb2.reshape(1, -1),
      qw1, qb1.reshape(1, -1), qw2, qb2.reshape(1, -1))
    return ypred, loss.reshape(())


# speedup vs baseline: 1.6343x; 1.3400x over previous
import functools
import jax
import jax.numpy as jnp
from jax.experimental import pallas as pl
from jax.experimental.pallas import tpu as pltpu

FEAT = 256
KWAY = 5
NSHOT = 5
K1 = 13
K2 = 2
MARGIN = 0.2
BIN = 8
NTOT = 100
NS = KWAY * NSHOT
NQ = NTOT - NS
L = KWAY + NQ
NC = 20
XT = 128


def _matmul(a, b):
    return jax.lax.dot_general(a, b, (((1,), (1,)), ((), ())),
                               preferred_element_type=jnp.float32)


def _safe_sqrt_d2(d2):
    d2 = jnp.maximum(d2, 0.0)
    safe = jnp.where(d2 < 1e-12, 1.0, d2)
    return jnp.where(d2 < 1e-12, 0.0, jnp.sqrt(safe))


def _topk_onehot_gather(dist, table, k):
    nrow, ncol = dist.shape
    col = jax.lax.broadcasted_iota(jnp.int32, (nrow, ncol), 1)
    out = []
    d = dist
    for _ in range(k):
        m = jnp.min(d, axis=1, keepdims=True)
        eq = d == m
        idx = jnp.min(jnp.where(eq, col, ncol), axis=1, keepdims=True)
        oh = col == idx
        out.append(jax.lax.dot_general(
            oh.astype(jnp.float32), table, (((1,), (0,)), ((), ())),
            preferred_element_type=jnp.float32))
        d = jnp.where(oh, 1e30, d)
    return out


def _mlp_combine(planes, w1_ref, b1_ref, w2_ref, b2_ref, nin, nhid):
    h = []
    for o in range(nhid):
        acc = b1_ref[0, o]
        for c in range(nin):
            acc = acc + w1_ref[o, c] * planes[c]
        h.append(jnp.maximum(acc, 0.0))
    z = []
    for c in range(nin):
        acc = b2_ref[0, c]
        for o in range(nhid):
            acc = acc + w2_ref[c, o] * h[o]
        z.append(acc)
    zmax = z[0]
    for c in range(1, nin):
        zmax = jnp.maximum(zmax, z[c])
    e = [jnp.exp(zc - zmax) for zc in z]
    s = e[0]
    for c in range(1, nin):
        s = s + e[c]
    out = planes[0] * (e[0] / s)
    for c in range(1, nin):
        out = out + planes[c] * (e[c] / s)
    return out


def _episode_kernel(inpt_ref, wqk_ref, bqk_ref,
                    sw1_ref, sb1_ref, sw2_ref, sb2_ref,
                    qw1_ref, qb1_ref, qw2_ref, qb2_ref,
                    ypred_ref, loss_ref,
                    q_s, k_s, v_s, s_s, e_s):
    b = pl.program_id(0)
    x = inpt_ref[0]

    feat = _matmul(x, wqk_ref[...]) + bqk_ref[...]
    q_s[...] = feat[:, :FEAT]
    k_s[...] = feat[:, FEAT:]

    for n in range(NTOT):
        p = jax.lax.dot_general(q_s[n:n + 1, :], k_s[n:n + 1, :],
                                (((0,), (0,)), ((), ())),
                                preferred_element_type=jnp.float32)
        e_s[n] = jnp.exp(p)

    for c in range(NTOT // NC):
        part = jnp.sum(e_s[c * NC:(c + 1) * NC], axis=0)
        if c == 0:
            s_s[...] = part
        else:
            s_s[...] = s_s[...] + part
    s_s[...] = 1.0 / s_s[...]

    for n in range(NTOT):
        xr = inpt_ref[0, n:n + 1, :]
        scaled = e_s[n] * s_s[...]
        v_s[n:n + 1, :] = xr + jax.lax.dot_general(
            xr, scaled, (((1,), (0,)), ((), ())),
            preferred_element_type=jnp.float32)

    sup = v_s[0:NS, :]
    que = v_s[pl.ds(NS, NQ), :]

    ns2 = jnp.sum(sup * sup, axis=1, keepdims=True)
    nq2 = jnp.sum(que * que, axis=1, keepdims=True)
    pn = jnp.sqrt(ns2) * jnp.transpose(jnp.sqrt(nq2))
    pn = jnp.maximum(pn, 1e-6)
    dist = -(_matmul(sup, que) / pn)

    nb = _topk_onehot_gather(dist, que, K1)
    sfeat = _mlp_combine([sup] + nb, sw1_ref, sb1_ref, sw2_ref, sb2_ref,
                         K1 + 1, 16)

    nb2 = _topk_onehot_gather(jnp.transpose(dist), sup, K2)
    qfeat = _mlp_combine([que] + nb2, qw1_ref, qb1_ref, qw2_ref, qb2_ref,
                         K2 + 1, 16)

    protos = jnp.concatenate(
        [jnp.mean(sfeat[5 * c:5 * c + 5, :], axis=0, keepdims=True)
         for c in range(KWAY)], axis=0)

    f = jnp.concatenate([protos, qfeat], axis=0)
    g = _matmul(f, f)
    na2 = jnp.sum(f * f, axis=1, keepdims=True)
    d2 = na2 + jnp.transpose(na2) - 2.0 * g
    dmat = _safe_sqrt_d2(d2)
    r = jax.lax.broadcasted_iota(jnp.int32, (L, L), 0)
    c = jax.lax.broadcasted_iota(jnp.int32, (L, L), 1)
    eye = r == c
    dii = jnp.sum(jnp.where(eye, dmat, 0.0), axis=1, keepdims=True)
    flm = jnp.where(eye, 0.0, jnp.maximum(MARGIN + dii - dmat, 0.0))
    tot = jnp.sum(flm)
    num = jnp.sum(jnp.where(flm != 0.0, 1.0, 0.0))
    mean_b = jnp.where(num == 0.0, 0.0, tot / jnp.where(num == 0.0, 1.0, num))

    np2 = jnp.sum(protos * protos, axis=1, keepdims=True)
    qf2 = jnp.sum(qfeat * qfeat, axis=1, keepdims=True)
    d2q = qf2 + jnp.transpose(np2) - 2.0 * _matmul(qfeat, protos)
    dq = _safe_sqrt_d2(d2q)

    @pl.when(b == 0)
    def _():
        ypred_ref[...] = dq
        loss_ref[...] = jnp.broadcast_to(mean_b / BIN, (1, 1))

    @pl.when(b > 0)
    def _():
        ypred_ref[...] = ypred_ref[...] + dq
        loss_ref[...] = loss_ref[...] + mean_b / BIN

    @pl.when(b == BIN - 1)
    def _():
        acc = -ypred_ref[...] / BIN
        m = jnp.max(acc, axis=1, keepdims=True)
        e = jnp.exp(acc - m)
        ypred_ref[...] = e / jnp.sum(e, axis=1, keepdims=True)


@jax.jit
def kernel(inpt, label, W_qk, b_qk, sw1, sb1, sw2, sb2, qw1, qb1, qw2, qb2):
    del label
    smem = functools.partial(pl.BlockSpec, memory_space=pltpu.SMEM)
    ypred, loss = pl.pallas_call(
        _episode_kernel,
        grid=(BIN,),
        in_specs=[
            pl.BlockSpec((1, NTOT, FEAT), lambda b: (b, 0, 0)),
            pl.BlockSpec((2 * FEAT, FEAT), lambda b: (0, 0)),
            pl.BlockSpec((1, 2 * FEAT), lambda b: (0, 0)),
            smem((16, K1 + 1), lambda b: (0, 0)),
            smem((1, 16), lambda b: (0, 0)),
            smem((K1 + 1, 16), lambda b: (0, 0)),
            smem((1, K1 + 1), lambda b: (0, 0)),
            smem((16, K2 + 1), lambda b: (0, 0)),
            smem((1, 16), lambda b: (0, 0)),
            smem((K2 + 1, 16), lambda b: (0, 0)),
            smem((1, K2 + 1), lambda b: (0, 0)),
        ],
        out_specs=[
            pl.BlockSpec((NQ, KWAY), lambda b: (0, 0)),
            pl.BlockSpec((1, 1), lambda b: (0, 0)),
        ],
        out_shape=[
            jax.ShapeDtypeStruct((NQ, KWAY), jnp.float32),
            jax.ShapeDtypeStruct((1, 1), jnp.float32),
        ],
        scratch_shapes=[
            pltpu.VMEM((NTOT, FEAT), jnp.float32),
            pltpu.VMEM((NTOT, FEAT), jnp.float32),
            pltpu.VMEM((NTOT, FEAT), jnp.float32),
            pltpu.VMEM((FEAT, FEAT), jnp.float32),
            pltpu.VMEM((NTOT, FEAT, FEAT), jnp.float32),
        ],
    )(inpt, W_qk, b_qk.reshape(1, -1),
      sw1, sb1.reshape(1, -1), sw2, sb2.reshape(1, -1),
      qw1, qb1.reshape(1, -1), qw2, qb2.reshape(1, -1))
    return ypred, loss.reshape(())
